# baseline (speedup 1.0000x reference)
import jax
import jax.numpy as jnp
from jax.experimental import pallas as pl
from jax.experimental.pallas import tpu as pltpu

N = 10000
NFEAT = 128
NHID = 128
NCLASS = 16
SHID = N // 3

BX = 200
BK2 = 1024
NB1 = N // BX
NB2 = pl.cdiv(N, BK2)
BM = 400
NBM = N // BM


def _a_kernel(x_ref, wg_ref, w1_ref, b1_ref, w2_ref, b2_ref,
              xw_ref, sc_ref, h_ref):
    i = pl.program_id(0)

    @pl.when(i < NB1)
    def _():
        x = x_ref[...]
        xw_ref[...] = jax.lax.dot_general(
            x, wg_ref[...], (((1,), (0,)), ((), ())),
            preferred_element_type=jnp.float32)
        s = jnp.sum(x, axis=1, keepdims=True)
        part = jnp.sum(w1_ref[...] * s, axis=0, keepdims=True)

        @pl.when(i == 0)
        def _():
            h_ref[...] = part

        @pl.when(i > 0)
        def _():
            h_ref[...] = h_ref[...] + part

        @pl.when(i == NB1 - 1)
        def _():
            h_ref[...] = jax.nn.relu(h_ref[...] + b1_ref[...])

    @pl.when(i >= NB1)
    def _():
        acc = jax.lax.dot_general(
            h_ref[...], w2_ref[...], (((1,), (0,)), ((), ())),
            preferred_element_type=jnp.float32)
        sc_ref[...] = jax.nn.sigmoid(acc + b2_ref[...])


def _bc_kernel(adj_ref, xw_ref, bg_ref, wcat_ref, score_ref, eps_ref,
               b11_ref, b12_ref, out_ref, m_ref):
    i = pl.program_id(0)

    @pl.when(i < NBM)
    def _():
        h = jax.lax.dot_general(
            adj_ref[...], xw_ref[...], (((1,), (0,)), ((), ())),
            preferred_element_type=jnp.float32)
        h = jax.nn.relu(h + bg_ref[...])
        hw = jax.lax.dot_general(
            h, wcat_ref[...], (((1,), (0,)), ((), ())),
            preferred_element_type=jnp.float32)
        m_ref[pl.ds(i * BM, BM), :] = hw * score_ref[pl.ds(i * BM, BM), :]

    @pl.when(i >= NBM)
    def _():
        acc = jax.lax.dot_general(
            adj_ref[...], m_ref[...], (((1,), (0,)), ((), ())),
            preferred_element_type=jnp.float32)
        mean = acc[:, :NCLASS] + b11_ref[...]
        logstd = acc[:, NCLASS:] + b12_ref[...]
        z = eps_ref[...] * jnp.exp(logstd) + mean
        zmax = jnp.max(z, axis=1, keepdims=True)
        ze = z - zmax
        out_ref[...] = ze - jnp.log(
            jnp.sum(jnp.exp(ze), axis=1, keepdims=True))


def kernel(x, adj, W_gc1, b_gc1, W_fc11, b_fc11, W_fc12, b_fc12,
           W_se1, b_se1, W_se2, b_se2, eps):
    f32 = jnp.float32

    xw, sc_row = pl.pallas_call(
        _a_kernel,
        grid=(NB1 + NB2,),
        in_specs=[
            pl.BlockSpec((BX, NFEAT), lambda i: (jnp.minimum(i, NB1 - 1), 0)),
            pl.BlockSpec((NFEAT, NHID), lambda i: (0, 0)),
            pl.BlockSpec((BX, SHID), lambda i: (jnp.minimum(i, NB1 - 1), 0)),
            pl.BlockSpec((1, SHID), lambda i: (0, 0)),
            pl.BlockSpec((SHID, BK2),
                         lambda i: (0, jnp.maximum(i - NB1, 0))),
            pl.BlockSpec((1, BK2), lambda i: (0, jnp.maximum(i - NB1, 0))),
        ],
        out_specs=[
            pl.BlockSpec((BX, NHID), lambda i: (jnp.minimum(i, NB1 - 1), 0)),
            pl.BlockSpec((1, BK2), lambda i: (0, jnp.maximum(i - NB1, 0))),
        ],
        out_shape=[
            jax.ShapeDtypeStruct((N, NHID), f32),
            jax.ShapeDtypeStruct((1, N), f32),
        ],
        scratch_shapes=[pltpu.VMEM((1, SHID), f32)],
        compiler_params=pltpu.CompilerParams(
            dimension_semantics=("arbitrary",)),
    )(x, W_gc1, W_se1, b_se1.reshape(1, SHID), W_se2, b_se2.reshape(1, N))

    score = sc_row.reshape(N, 1)
    wcat = jnp.concatenate([W_fc11, W_fc12], axis=1)

    out = pl.pallas_call(
        _bc_kernel,
        grid=(2 * NBM,),
        in_specs=[
            pl.BlockSpec((BM, N), lambda i: (jax.lax.rem(i, NBM), 0)),
            pl.BlockSpec((N, NHID), lambda i: (0, 0)),
            pl.BlockSpec((1, NHID), lambda i: (0, 0)),
            pl.BlockSpec((NHID, 2 * NCLASS), lambda i: (0, 0)),
            pl.BlockSpec((N, 1), lambda i: (0, 0)),
            pl.BlockSpec((BM, NCLASS), lambda i: (jax.lax.rem(i, NBM), 0)),
            pl.BlockSpec((1, NCLASS), lambda i: (0, 0)),
            pl.BlockSpec((1, NCLASS), lambda i: (0, 0)),
        ],
        out_specs=pl.BlockSpec((BM, NCLASS),
                               lambda i: (jax.lax.rem(i, NBM), 0)),
        out_shape=jax.ShapeDtypeStruct((N, NCLASS), f32),
        scratch_shapes=[pltpu.VMEM((N, 2 * NCLASS), f32)],
        compiler_params=pltpu.CompilerParams(
            dimension_semantics=("arbitrary",)),
    )(adj, xw, b_gc1.reshape(1, NHID), wcat, score, eps,
      b_fc11.reshape(1, NCLASS), b_fc12.reshape(1, NCLASS))

    return out


# speedup vs baseline: 1.0138x; 1.0138x over previous
import jax
import jax.numpy as jnp
from jax.experimental import pallas as pl
from jax.experimental.pallas import tpu as pltpu

N = 10000
NFEAT = 128
NHID = 128
NCLASS = 16
SHID = N // 3

BX = 1000
NB1 = N // BX
BH = 384
NB2 = pl.cdiv(SHID, BH)
BM = 400
NBM = N // BM


def _a1_kernel(x_ref, wg_ref, w1_ref, b1_ref, xw_ref, h_ref, hacc_ref):
    i = pl.program_id(0)
    x = x_ref[...]
    xw_ref[...] = jax.lax.dot_general(
        x, wg_ref[...], (((1,), (0,)), ((), ())),
        preferred_element_type=jnp.float32)
    s = jnp.sum(x, axis=1, keepdims=True)
    part = jnp.sum(w1_ref[...] * s, axis=0, keepdims=True)

    @pl.when(i == 0)
    def _():
        hacc_ref[...] = part

    @pl.when(i > 0)
    def _():
        hacc_ref[...] = hacc_ref[...] + part

    @pl.when(i == NB1 - 1)
    def _():
        h_ref[...] = jax.nn.relu(hacc_ref[...] + b1_ref[...])


def _a2_kernel(h_ref, w2_ref, b2_ref, sc_ref):
    i = pl.program_id(0)
    row = jax.lax.broadcasted_iota(jnp.int32, (BH, 1), 0) + i * BH
    prod = jnp.where(row < SHID, w2_ref[...] * h_ref[...], 0.0)
    part = jnp.sum(prod, axis=0, keepdims=True)

    @pl.when(i == 0)
    def _():
        sc_ref[...] = part

    @pl.when(i > 0)
    def _():
        sc_ref[...] = sc_ref[...] + part

    @pl.when(i == NB2 - 1)
    def _():
        sc_ref[...] = jax.nn.sigmoid(sc_ref[...] + b2_ref[...])


def _bc_kernel(adj_ref, xw_ref, bg_ref, wcat_ref, score_ref, eps_ref,
               b11_ref, b12_ref, out_ref, m_ref):
    i = pl.program_id(0)

    @pl.when(i < NBM)
    def _():
        h = jax.lax.dot_general(
            adj_ref[...], xw_ref[...], (((1,), (0,)), ((), ())),
            preferred_element_type=jnp.float32)
        h = jax.nn.relu(h + bg_ref[...])
        hw = jax.lax.dot_general(
            h, wcat_ref[...], (((1,), (0,)), ((), ())),
            preferred_element_type=jnp.float32)
        m_ref[pl.ds(i * BM, BM), :] = hw * score_ref[pl.ds(i * BM, BM), :]

    @pl.when(i >= NBM)
    def _():
        acc = jax.lax.dot_general(
            adj_ref[...], m_ref[...], (((1,), (0,)), ((), ())),
            preferred_element_type=jnp.float32)
        mean = acc[:, :NCLASS] + b11_ref[...]
        logstd = acc[:, NCLASS:] + b12_ref[...]
        z = eps_ref[...] * jnp.exp(logstd) + mean
        zmax = jnp.max(z, axis=1, keepdims=True)
        ze = z - zmax
        out_ref[...] = ze - jnp.log(
            jnp.sum(jnp.exp(ze), axis=1, keepdims=True))


def kernel(x, adj, W_gc1, b_gc1, W_fc11, b_fc11, W_fc12, b_fc12,
           W_se1, b_se1, W_se2, b_se2, eps):
    f32 = jnp.float32

    xw, h = pl.pallas_call(
        _a1_kernel,
        grid=(NB1,),
        in_specs=[
            pl.BlockSpec((BX, NFEAT), lambda i: (i, 0)),
            pl.BlockSpec((NFEAT, NHID), lambda i: (0, 0)),
            pl.BlockSpec((BX, SHID), lambda i: (i, 0)),
            pl.BlockSpec((1, SHID), lambda i: (0, 0)),
        ],
        out_specs=[
            pl.BlockSpec((BX, NHID), lambda i: (i, 0)),
            pl.BlockSpec((1, SHID), lambda i: (0, 0)),
        ],
        out_shape=[
            jax.ShapeDtypeStruct((N, NHID), f32),
            jax.ShapeDtypeStruct((1, SHID), f32),
        ],
        scratch_shapes=[pltpu.VMEM((1, SHID), f32)],
        compiler_params=pltpu.CompilerParams(
            dimension_semantics=("arbitrary",)),
    )(x, W_gc1, W_se1, b_se1.reshape(1, SHID))

    h_col = h.reshape(SHID, 1)

    sc_row = pl.pallas_call(
        _a2_kernel,
        grid=(NB2,),
        in_specs=[
            pl.BlockSpec((BH, 1), lambda i: (i, 0)),
            pl.BlockSpec((BH, N), lambda i: (i, 0)),
            pl.BlockSpec((1, N), lambda i: (0, 0)),
        ],
        out_specs=pl.BlockSpec((1, N), lambda i: (0, 0)),
        out_shape=jax.ShapeDtypeStruct((1, N), f32),
        compiler_params=pltpu.CompilerParams(
            dimension_semantics=("arbitrary",)),
    )(h_col, W_se2, b_se2.reshape(1, N))

    score = sc_row.reshape(N, 1)
    wcat = jnp.concatenate([W_fc11, W_fc12], axis=1)

    out = pl.pallas_call(
        _bc_kernel,
        grid=(2 * NBM,),
        in_specs=[
            pl.BlockSpec((BM, N), lambda i: (jax.lax.rem(i, NBM), 0)),
            pl.BlockSpec((N, NHID), lambda i: (0, 0)),
            pl.BlockSpec((1, NHID), lambda i: (0, 0)),
            pl.BlockSpec((NHID, 2 * NCLASS), lambda i: (0, 0)),
            pl.BlockSpec((N, 1), lambda i: (0, 0)),
            pl.BlockSpec((BM, NCLASS), lambda i: (jax.lax.rem(i, NBM), 0)),
            pl.BlockSpec((1, NCLASS), lambda i: (0, 0)),
            pl.BlockSpec((1, NCLASS), lambda i: (0, 0)),
        ],
        out_specs=pl.BlockSpec((BM, NCLASS),
                               lambda i: (jax.lax.rem(i, NBM), 0)),
        out_shape=jax.ShapeDtypeStruct((N, NCLASS), f32),
        scratch_shapes=[pltpu.VMEM((N, 2 * NCLASS), f32)],
        compiler_params=pltpu.CompilerParams(
            dimension_semantics=("arbitrary",)),
    )(adj, xw, b_gc1.reshape(1, NHID), wcat, score, eps,
      b_fc11.reshape(1, NCLASS), b_fc12.reshape(1, NCLASS))

    return out
